# baseline (speedup 1.0000x reference)
import functools

import jax
import jax.numpy as jnp
from jax import lax
from jax.experimental import pallas as pl
from jax.experimental.pallas import tpu as pltpu
from jax.experimental.pallas import tpu_sc as plsc

N_GENE, N_DRUG, N_CELL = 10000, 2000, 1000
D, EE, MID, BATCH = 128, 256, 512, 4096

NC, NS, NW = 2, 16, 32
K = 128

GP, DP, CP = 10240, 2048, 1024
AG = 10240
AD = 2048
HG, HD, HC = N_GENE + 8, N_DRUG + 8, N_CELL + 8

EP_GG, EP_DG, EP_GD, EP_CG = 327680, 65536, 65536, 32768

_MESH = plsc.VectorSubcoreMesh(core_axis_name="c", subcore_axis_name="s",
                               num_cores=NC, num_subcores=NS)

_Z16F = functools.partial(jnp.zeros, (16,), jnp.float32)


def _leaky(x):
    return jnp.maximum(x, 0.01 * x)


def _deg_body(sgg, dgg, sdg, ddg, sgd, dgd, scg, dcg,
              o_gg_s, o_gg_d, o_dg_s, o_dg_d, o_gd_s, o_gd_d, o_cg_s, o_cg_d,
              stage, stage2, t_gg_s, t_gg_d, t_dg_s, t_dg_d, t_gd_s, t_gd_d,
              t_cg_s, t_cg_d, ssem0, ssem1):
    wid = lax.axis_index("c") * NS + lax.axis_index("s")
    ones = jnp.full((16,), 1.0, jnp.float32)

    def zero(tbl, n):
        def b(i, _):
            tbl[pl.ds(i * 16, 16)] = _Z16F()
            return ()
        lax.fori_loop(0, n // 16, b, ())

    for tbl, n in ((t_gg_s, GP), (t_gg_d, GP), (t_dg_s, DP), (t_dg_d, GP),
                   (t_gd_s, GP), (t_gd_d, DP), (t_cg_s, CP), (t_cg_d, GP)):
        zero(tbl, n)

    work = [(sgg, t_gg_s, EP_GG), (dgg, t_gg_d, EP_GG),
            (sdg, t_dg_s, EP_DG), (ddg, t_dg_d, EP_DG),
            (sgd, t_gd_s, EP_GD), (dgd, t_gd_d, EP_GD),
            (scg, t_cg_s, EP_CG), (dcg, t_cg_d, EP_CG)]
    stages = [stage, stage2]
    sems = [ssem0, ssem1]

    def start_stage(r):
        idx_hbm, _, ep = work[r]
        epw = ep // NW
        return pltpu.async_copy(idx_hbm.at[pl.ds(wid * epw, epw)],
                                stages[r & 1].at[pl.ds(0, epw)], sems[r & 1])

    d = start_stage(0)
    for r in range(8):
        nxt = start_stage(r + 1) if r + 1 < 8 else None
        d.wait()
        _, tbl, ep = work[r]
        stg = stages[r & 1]

        def b(i, _, tbl=tbl, stg=stg):
            for u in range(4):
                v = stg[pl.ds(i * 64 + u * 16, 16)]
                plsc.addupdate_scatter(tbl, [v], ones)
            return ()
        lax.fori_loop(0, ep // NW // 64, b, ())
        d = nxt

    for tbl, out in ((t_gg_s, o_gg_s), (t_gg_d, o_gg_d), (t_dg_s, o_dg_s),
                     (t_dg_d, o_dg_d), (t_gd_s, o_gd_s), (t_gd_d, o_gd_d),
                     (t_cg_s, o_cg_s), (t_cg_d, o_cg_d)):
        pltpu.sync_copy(tbl, out.at[wid])


_SC_PARAMS = pltpu.CompilerParams(needs_layout_passes=False)

_deg_kernel = functools.partial(
    pl.kernel, _deg_body, mesh=_MESH, compiler_params=_SC_PARAMS,
    out_type=[jax.ShapeDtypeStruct((NW, GP), jnp.float32),
              jax.ShapeDtypeStruct((NW, GP), jnp.float32),
              jax.ShapeDtypeStruct((NW, DP), jnp.float32),
              jax.ShapeDtypeStruct((NW, GP), jnp.float32),
              jax.ShapeDtypeStruct((NW, GP), jnp.float32),
              jax.ShapeDtypeStruct((NW, DP), jnp.float32),
              jax.ShapeDtypeStruct((NW, CP), jnp.float32),
              jax.ShapeDtypeStruct((NW, GP), jnp.float32)],
    scratch_types=[pltpu.VMEM((EP_GG // NW,), jnp.int32),
                   pltpu.VMEM((EP_GG // NW,), jnp.int32),
                   pltpu.VMEM((GP,), jnp.float32),
                   pltpu.VMEM((GP,), jnp.float32),
                   pltpu.VMEM((DP,), jnp.float32),
                   pltpu.VMEM((GP,), jnp.float32),
                   pltpu.VMEM((GP,), jnp.float32),
                   pltpu.VMEM((DP,), jnp.float32),
                   pltpu.VMEM((CP,), jnp.float32),
                   pltpu.VMEM((GP,), jnp.float32),
                   pltpu.SemaphoreType.DMA,
                   pltpu.SemaphoreType.DMA],
)()


def _norm(partials, n):
    deg = jnp.sum(partials, axis=0)[:n]
    return lax.rsqrt(jnp.maximum(deg, 1.0))


def _dense1_body(cf, df, ge, cW, cb, dW, db, eW, eb, mW1,
                 pgg_s, pdg_s, pcg_s, pgd_s,
                 h_gg, h_dg, h_cg, h_gd, drug_enc_o, cell_mid):
    cell_enc = _leaky(jnp.dot(cf[...], cW[...],
                              preferred_element_type=jnp.float32) + cb[...])
    drug_enc = _leaky(jnp.dot(df[...], dW[...],
                              preferred_element_type=jnp.float32) + db[...])
    gene = ge[...]

    h_gg[: N_GENE, :] = gene * _norm(pgg_s[...], N_GENE)[:, None]
    h_gg[N_GENE:, :] = jnp.zeros((8, D), jnp.float32)
    h_gd[: N_GENE, :] = gene * _norm(pgd_s[...], N_GENE)[:, None]
    h_gd[N_GENE:, :] = jnp.zeros((8, D), jnp.float32)
    h_dg[: N_DRUG, :] = drug_enc * _norm(pdg_s[...], N_DRUG)[:, None]
    h_dg[N_DRUG:, :] = jnp.zeros((8, D), jnp.float32)
    h_cg[: N_CELL, :] = cell_enc * _norm(pcg_s[...], N_CELL)[:, None]
    h_cg[N_CELL:, :] = jnp.zeros((8, D), jnp.float32)
    drug_enc_o[...] = drug_enc

    expr = _leaky(jnp.dot(cf[...], eW[...],
                          preferred_element_type=jnp.float32) + eb[...])
    cell_mid[...] = jnp.dot(expr, mW1[...], preferred_element_type=jnp.float32)


def _dense1(cf, df, ge, cW, cb, dW, db, eW, eb, mW1, pgg_s, pdg_s, pcg_s, pgd_s):
    return pl.pallas_call(
        _dense1_body,
        out_shape=[jax.ShapeDtypeStruct((HG, D), jnp.float32),
                   jax.ShapeDtypeStruct((HD, D), jnp.float32),
                   jax.ShapeDtypeStruct((HC, D), jnp.float32),
                   jax.ShapeDtypeStruct((HG, D), jnp.float32),
                   jax.ShapeDtypeStruct((N_DRUG, D), jnp.float32),
                   jax.ShapeDtypeStruct((N_CELL, MID), jnp.float32)],
    )(cf, df, ge, cW, cb, dW, db, eW, eb, mW1, pgg_s, pdg_s, pcg_s, pgd_s)


_ZR = 16
_SMAX = 40
_GS = 2
_KG = K // _GS


def _zero_zbuf(zbuf):
    def b(r, _):
        for j in range(8):
            zbuf[r, pl.ds(j * 16, 16)] = _Z16F()
        return ()
    lax.fori_loop(0, _ZR, b, ())


def _do_rel(c, s, src2d, dst2d, h_hbm, spmem, out_hbm,
            sstage, dstage, pays, gsems, ssems, zsem, zbuf, ep, n_row_chunks):
    zd = []
    for k in range(n_row_chunks * (K // _ZR)):
        r0 = s * n_row_chunks * K + k * _ZR
        zd.append(pltpu.async_copy(zbuf, spmem.at[pl.ds(r0, _ZR)], zsem))
    for d in zd:
        d.wait()
    plsc.subcore_barrier()

    wid = c * NS + s
    nch = ep // NW // K
    srounds = max(nch // _SMAX, 1)
    spr = nch // srounds

    def start_gather(i, b):
        return [pltpu.async_copy(
                    h_hbm.at[sstage.at[i, pl.ds(g * _KG, _KG)]],
                    pays[b].at[pl.ds(g * _KG, _KG)], gsems[b])
                for g in range(_GS)]

    for rnd in range(srounds):
        row0 = wid * nch + rnd * spr
        pltpu.sync_copy(src2d.at[pl.ds(row0, spr), :],
                        sstage.at[pl.ds(0, spr), :])
        pltpu.sync_copy(dst2d.at[pl.ds(row0, spr), :],
                        dstage.at[pl.ds(0, spr), :])
        gd = [None, None]
        sd = [None, None]
        for i in range(spr):
            b = i & 1
            if sd[b] is not None:
                sd[b].wait()
            gd[b] = start_gather(i, b)
            if i > 0:
                pb = (i - 1) & 1
                for d in gd[pb]:
                    d.wait()
                sd[pb] = pltpu.async_copy(pays[pb],
                                          spmem.at[dstage.at[i - 1]],
                                          ssems[pb], add=True)
        lb = (spr - 1) & 1
        for d in gd[lb]:
            d.wait()
        sd[lb] = pltpu.async_copy(pays[lb], spmem.at[dstage.at[spr - 1]],
                                  ssems[lb], add=True)
        for b in range(2):
            if sd[b] is not None:
                sd[b].wait()
    plsc.subcore_barrier()

    dd = []
    for k in range(n_row_chunks):
        r0 = (s * n_row_chunks + k) * K
        dd.append(pltpu.async_copy(spmem.at[pl.ds(r0, K)],
                                   out_hbm.at[c, pl.ds(r0, K)], zsem))
    for d in dd:
        d.wait()
    plsc.subcore_barrier()


def _l1_body(sgg, dgg, sdg, ddg, sgd, dgd, scg, dcg, h_gg, h_dg, h_cg, h_gd,
             p_gg, p_dg, p_cg, p_gd,
             sstage, dstage, pay0, pay1, zbuf, spm_g,
             semg0, semg1, sems0, sems1, zsem):
    c = lax.axis_index("c")
    s = lax.axis_index("s")
    pays, gsems, ssems = [pay0, pay1], [semg0, semg1], [sems0, sems1]
    _zero_zbuf(zbuf)
    _do_rel(c, s, sgd, dgd, h_gd, spm_g, p_gd, sstage, dstage, pays,
            gsems, ssems, zsem, zbuf, EP_GD, AD // (NS * K))
    _do_rel(c, s, sgg, dgg, h_gg, spm_g, p_gg, sstage, dstage, pays,
            gsems, ssems, zsem, zbuf, EP_GG, AG // (NS * K))
    _do_rel(c, s, sdg, ddg, h_dg, spm_g, p_dg, sstage, dstage, pays,
            gsems, ssems, zsem, zbuf, EP_DG, AG // (NS * K))
    _do_rel(c, s, scg, dcg, h_cg, spm_g, p_cg, sstage, dstage, pays,
            gsems, ssems, zsem, zbuf, EP_CG, AG // (NS * K))


_SC_EDGE_SCRATCH = [pltpu.VMEM((_SMAX, K), jnp.int32),
                    pltpu.VMEM((_SMAX, K), jnp.int32),
                    pltpu.VMEM((K, D), jnp.float32),
                    pltpu.VMEM((K, D), jnp.float32),
                    pltpu.VMEM((_ZR, D), jnp.float32)]

_l1_kernel = functools.partial(
    pl.kernel, _l1_body, mesh=_MESH, compiler_params=_SC_PARAMS,
    out_type=[jax.ShapeDtypeStruct((NC, AG, D), jnp.float32),
              jax.ShapeDtypeStruct((NC, AG, D), jnp.float32),
              jax.ShapeDtypeStruct((NC, AG, D), jnp.float32),
              jax.ShapeDtypeStruct((NC, AD, D), jnp.float32)],
    scratch_types=_SC_EDGE_SCRATCH
    + [pltpu.VMEM_SHARED((AG, D), jnp.float32),
       pltpu.SemaphoreType.DMA, pltpu.SemaphoreType.DMA,
       pltpu.SemaphoreType.DMA, pltpu.SemaphoreType.DMA,
       pltpu.SemaphoreType.DMA],
)()


def _l2_body(sgd, dgd, h1g, p2_gd, sstage, dstage, pay0, pay1, zbuf, spm_d,
             semg0, semg1, sems0, sems1, zsem):
    c = lax.axis_index("c")
    s = lax.axis_index("s")
    _zero_zbuf(zbuf)
    _do_rel(c, s, sgd, dgd, h1g, spm_d, p2_gd, sstage, dstage,
            [pay0, pay1], [semg0, semg1], [sems0, sems1], zsem, zbuf,
            EP_GD, AD // (NS * K))


_l2_kernel = functools.partial(
    pl.kernel, _l2_body, mesh=_MESH, compiler_params=_SC_PARAMS,
    out_type=[jax.ShapeDtypeStruct((NC, AD, D), jnp.float32)],
    scratch_types=_SC_EDGE_SCRATCH
    + [pltpu.VMEM_SHARED((AD, D), jnp.float32),
       pltpu.SemaphoreType.DMA, pltpu.SemaphoreType.DMA,
       pltpu.SemaphoreType.DMA, pltpu.SemaphoreType.DMA,
       pltpu.SemaphoreType.DMA],
)()


def _comb1_body(p_gg, p_dg, p_cg, p_gd, pgg_d, pdg_d, pcg_d, pgd_s, pgd_d,
                Wgg, bgg, Wdg, bdg, Wcg, bcg, Wgd, bgd, ge, drug_enc,
                h1g_s, h1_drug_o):
    def agg(p, n, nd_part):
        a = (p[0, :n, :] + p[1, :n, :]) * _norm(nd_part[...], n)[:, None]
        return a

    out_g = jnp.dot(agg(p_gg, N_GENE, pgg_d), Wgg[...],
                    preferred_element_type=jnp.float32) + bgg[...]
    out_g = out_g + jnp.dot(agg(p_dg, N_GENE, pdg_d), Wdg[...],
                            preferred_element_type=jnp.float32) + bdg[...]
    out_g = out_g + jnp.dot(agg(p_cg, N_GENE, pcg_d), Wcg[...],
                            preferred_element_type=jnp.float32) + bcg[...]
    h1_gene = _leaky(out_g + 0.5 * ge[...])
    h1g_s[: N_GENE, :] = h1_gene * _norm(pgd_s[...], N_GENE)[:, None]
    h1g_s[N_GENE:, :] = jnp.zeros((8, D), jnp.float32)

    out_d = jnp.dot(agg(p_gd, N_DRUG, pgd_d), Wgd[...],
                    preferred_element_type=jnp.float32) + bgd[...]
    h1_drug_o[...] = _leaky(out_d + 0.5 * drug_enc[...])


def _comb1(p_gg, p_dg, p_cg, p_gd, pgg_d, pdg_d, pcg_d, pgd_s, pgd_d,
           Wgg, bgg, Wdg, bdg, Wcg, bcg, Wgd, bgd, ge, drug_enc):
    return pl.pallas_call(
        _comb1_body,
        out_shape=[jax.ShapeDtypeStruct((HG, D), jnp.float32),
                   jax.ShapeDtypeStruct((N_DRUG, D), jnp.float32)],
    )(p_gg, p_dg, p_cg, p_gd, pgg_d, pdg_d, pcg_d, pgd_s, pgd_d,
      Wgg, bgg, Wdg, bdg, Wcg, bcg, Wgd, bgd, ge, drug_enc)


def _comb2_body(p2_gd, pgd_d, W2, b2, h1_drug, mW2, drug_mid):
    a = (p2_gd[0, :N_DRUG, :] + p2_gd[1, :N_DRUG, :]) \
        * _norm(pgd_d[...], N_DRUG)[:, None]
    h2 = _leaky(jnp.dot(a, W2[...], preferred_element_type=jnp.float32)
                + b2[...] + 0.5 * h1_drug[...])
    drug_mid[...] = jnp.dot(h2, mW2[...], preferred_element_type=jnp.float32)


def _comb2(p2_gd, pgd_d, W2, b2, h1_drug, mW2):
    return pl.pallas_call(
        _comb2_body,
        out_shape=jax.ShapeDtypeStruct((N_DRUG, MID), jnp.float32),
    )(p2_gd, pgd_d, W2, b2, h1_drug, mW2)


_HB = 64


def _head_body(cell_mid, drug_mid, cidx_hbm, didx_hbm, x_out,
               cidx, didx, bufc, bufd, semc, semd):
    wid = lax.axis_index("c") * NS + lax.axis_index("s")
    for ch in range(2):
        base = wid * 128 + ch * _HB
        pltpu.sync_copy(cidx_hbm.at[pl.ds(base, _HB)], cidx)
        pltpu.sync_copy(didx_hbm.at[pl.ds(base, _HB)], didx)
        cpc = pltpu.async_copy(cell_mid.at[cidx], bufc, semc)
        cpd = pltpu.async_copy(drug_mid.at[didx], bufd, semd)
        cpc.wait()
        cpd.wait()

        def addrow(r, _):
            for j in range(MID // 16):
                sl = pl.ds(j * 16, 16)
                bufc[r, sl] = bufc[r, sl] + bufd[r, sl]
            return ()
        lax.fori_loop(0, _HB, addrow, ())
        pltpu.sync_copy(bufc, x_out.at[pl.ds(base, _HB)])


_head_kernel = functools.partial(
    pl.kernel, _head_body, mesh=_MESH,
    out_type=[jax.ShapeDtypeStruct((BATCH, MID), jnp.float32)],
    scratch_types=[pltpu.VMEM((_HB,), jnp.int32),
                   pltpu.VMEM((_HB,), jnp.int32),
                   pltpu.VMEM((_HB, MID), jnp.float32),
                   pltpu.VMEM((_HB, MID), jnp.float32),
                   pltpu.SemaphoreType.DMA,
                   pltpu.SemaphoreType.DMA],
)()


def _out_body(x, mb, oW, ob, out):
    h = _leaky(x[...] + mb[...])
    out[...] = jnp.dot(h, oW[...], preferred_element_type=jnp.float32) + ob[...]


def _out_head(x, mb, oW, ob):
    return pl.pallas_call(
        _out_body,
        out_shape=jax.ShapeDtypeStruct((BATCH, 1), jnp.float32),
    )(x, mb, oW, ob)


def _pad_edges(ei, ns, nd, ep, spare_d):
    e = ei.shape[1]
    ar = jnp.arange(ep - e, dtype=jnp.int32)
    src = jnp.concatenate([ei[0], ns + ar % 8])
    dst = jnp.concatenate([ei[1], nd + ar % spare_d])
    return src, dst


def kernel(drug_features, cell_features, cell_index, drug_index, gene_index,
           gg_edge_index, dg_edge_index, gd_edge_index, cg_edge_index,
           gc_edge_index, params):
    p = params
    sgg, dgg = _pad_edges(gg_edge_index, N_GENE, N_GENE, EP_GG, AG - N_GENE)
    sdg, ddg = _pad_edges(dg_edge_index, N_DRUG, N_GENE, EP_DG, AG - N_GENE)
    sgd, dgd = _pad_edges(gd_edge_index, N_GENE, N_DRUG, EP_GD, AD - N_DRUG)
    scg, dcg = _pad_edges(cg_edge_index, N_CELL, N_GENE, EP_CG, AG - N_GENE)

    (pgg_s, pgg_d, pdg_s, pdg_d, pgd_s, pgd_d, pcg_s, pcg_d) = _deg_kernel(
        sgg, dgg, sdg, ddg, sgd, dgd, scg, dcg)

    h_gg, h_dg, h_cg, h_gd, drug_enc, cell_mid = _dense1(
        cell_features, drug_features, p["gene_emb"],
        p["cell_enc_W"], p["cell_enc_b"], p["drug_enc_W"], p["drug_enc_b"],
        p["expr_enc_W"], p["expr_enc_b"], p["mid_W"][:EE],
        pgg_s, pdg_s, pcg_s, pgd_s)

    r2 = lambda a: a.reshape(-1, K)
    p_gg, p_dg, p_cg, p_gd = _l1_kernel(
        r2(sgg), r2(dgg), r2(sdg), r2(ddg), r2(sgd), r2(dgd),
        r2(scg), r2(dcg), h_gg, h_dg, h_cg, h_gd)

    h1g_s, h1_drug = _comb1(
        p_gg, p_dg, p_cg, p_gd, pgg_d, pdg_d, pcg_d, pgd_s, pgd_d,
        p["W1_gg"], p["b1_gg"], p["W1_dg"], p["b1_dg"],
        p["W1_cg"], p["b1_cg"], p["W1_gd"], p["b1_gd"],
        p["gene_emb"], drug_enc)

    (p2_gd,) = _l2_kernel(r2(sgd), r2(dgd), h1g_s)

    drug_mid = _comb2(p2_gd, pgd_d, p["W2_gd"], p["b2_gd"], h1_drug,
                      p["mid_W"][EE:])

    (x_sum,) = _head_kernel(cell_mid, drug_mid, cell_index, drug_index)

    return _out_head(x_sum, p["mid_b"], p["out_W"], p["out_b"])


# speedup vs baseline: 1.0451x; 1.0451x over previous
import functools

import jax
import jax.numpy as jnp
from jax import lax
from jax.experimental import pallas as pl
from jax.experimental.pallas import tpu as pltpu
from jax.experimental.pallas import tpu_sc as plsc

N_GENE, N_DRUG, N_CELL = 10000, 2000, 1000
D, EE, MID, BATCH = 128, 256, 512, 4096

NC, NS, NW = 2, 16, 32
K = 128

GP, DP, CP = 10240, 2048, 1024
AG = 10240
AD = 2048
HG, HD, HC = N_GENE + 8, N_DRUG + 8, N_CELL + 8

EP_GG, EP_DG, EP_GD, EP_CG = 327680, 65536, 65536, 32768

_MESH = plsc.VectorSubcoreMesh(core_axis_name="c", subcore_axis_name="s",
                               num_cores=NC, num_subcores=NS)

_Z16F = functools.partial(jnp.zeros, (16,), jnp.float32)


def _leaky(x):
    return jnp.maximum(x, 0.01 * x)


def _deg_body(sgg, dgg, sdg, ddg, sgd, dgd, scg, dcg,
              o_gg_s, o_gg_d, o_dg_s, o_dg_d, o_gd_s, o_gd_d, o_cg_s, o_cg_d,
              stage, stage2, t_gg_s, t_gg_d, t_dg_s, t_dg_d, t_gd_s, t_gd_d,
              t_cg_s, t_cg_d, ssem0, ssem1):
    wid = lax.axis_index("c") * NS + lax.axis_index("s")
    ones = jnp.full((16,), 1.0, jnp.float32)

    def zero(tbl, n):
        def b(i, _):
            tbl[pl.ds(i * 16, 16)] = _Z16F()
            return ()
        lax.fori_loop(0, n // 16, b, ())

    for tbl, n in ((t_gg_s, GP), (t_gg_d, GP), (t_dg_s, DP), (t_dg_d, GP),
                   (t_gd_s, GP), (t_gd_d, DP), (t_cg_s, CP), (t_cg_d, GP)):
        zero(tbl, n)

    work = [(sgg, t_gg_s, EP_GG), (dgg, t_gg_d, EP_GG),
            (sdg, t_dg_s, EP_DG), (ddg, t_dg_d, EP_DG),
            (sgd, t_gd_s, EP_GD), (dgd, t_gd_d, EP_GD),
            (scg, t_cg_s, EP_CG), (dcg, t_cg_d, EP_CG)]
    stages = [stage, stage2]
    sems = [ssem0, ssem1]

    def start_stage(r):
        idx_hbm, _, ep = work[r]
        epw = ep // NW
        return pltpu.async_copy(idx_hbm.at[pl.ds(wid * epw, epw)],
                                stages[r & 1].at[pl.ds(0, epw)], sems[r & 1])

    d = start_stage(0)
    for r in range(8):
        nxt = start_stage(r + 1) if r + 1 < 8 else None
        d.wait()
        _, tbl, ep = work[r]
        stg = stages[r & 1]

        def b(i, _, tbl=tbl, stg=stg):
            for u in range(4):
                v = stg[pl.ds(i * 64 + u * 16, 16)]
                plsc.addupdate_scatter(tbl, [v], ones)
            return ()
        lax.fori_loop(0, ep // NW // 64, b, ())
        d = nxt

    for tbl, out in ((t_gg_s, o_gg_s), (t_gg_d, o_gg_d), (t_dg_s, o_dg_s),
                     (t_dg_d, o_dg_d), (t_gd_s, o_gd_s), (t_gd_d, o_gd_d),
                     (t_cg_s, o_cg_s), (t_cg_d, o_cg_d)):
        pltpu.sync_copy(tbl, out.at[wid])


_SC_PARAMS = pltpu.CompilerParams(needs_layout_passes=False)

_deg_kernel = functools.partial(
    pl.kernel, _deg_body, mesh=_MESH, compiler_params=_SC_PARAMS,
    out_type=[jax.ShapeDtypeStruct((NW, GP), jnp.float32),
              jax.ShapeDtypeStruct((NW, GP), jnp.float32),
              jax.ShapeDtypeStruct((NW, DP), jnp.float32),
              jax.ShapeDtypeStruct((NW, GP), jnp.float32),
              jax.ShapeDtypeStruct((NW, GP), jnp.float32),
              jax.ShapeDtypeStruct((NW, DP), jnp.float32),
              jax.ShapeDtypeStruct((NW, CP), jnp.float32),
              jax.ShapeDtypeStruct((NW, GP), jnp.float32)],
    scratch_types=[pltpu.VMEM((EP_GG // NW,), jnp.int32),
                   pltpu.VMEM((EP_GG // NW,), jnp.int32),
                   pltpu.VMEM((GP,), jnp.float32),
                   pltpu.VMEM((GP,), jnp.float32),
                   pltpu.VMEM((DP,), jnp.float32),
                   pltpu.VMEM((GP,), jnp.float32),
                   pltpu.VMEM((GP,), jnp.float32),
                   pltpu.VMEM((DP,), jnp.float32),
                   pltpu.VMEM((CP,), jnp.float32),
                   pltpu.VMEM((GP,), jnp.float32),
                   pltpu.SemaphoreType.DMA,
                   pltpu.SemaphoreType.DMA],
)()


def _norm(partials, n):
    deg = jnp.sum(partials, axis=0)[:n]
    return lax.rsqrt(jnp.maximum(deg, 1.0))


def _enc_body(cf, df, cW, cb, dW, db, eW, eb, mW1,
              cell_enc_o, drug_enc_o, cell_mid):
    cell_enc_o[...] = _leaky(jnp.dot(cf[...], cW[...],
                                     preferred_element_type=jnp.float32)
                             + cb[...])
    drug_enc_o[...] = _leaky(jnp.dot(df[...], dW[...],
                                     preferred_element_type=jnp.float32)
                             + db[...])
    expr = _leaky(jnp.dot(cf[...], eW[...],
                          preferred_element_type=jnp.float32) + eb[...])
    cell_mid[...] = jnp.dot(expr, mW1[...], preferred_element_type=jnp.float32)


def _encoders(cf, df, cW, cb, dW, db, eW, eb, mW1):
    return pl.pallas_call(
        _enc_body,
        out_shape=[jax.ShapeDtypeStruct((N_CELL, D), jnp.float32),
                   jax.ShapeDtypeStruct((N_DRUG, D), jnp.float32),
                   jax.ShapeDtypeStruct((N_CELL, MID), jnp.float32)],
    )(cf, df, cW, cb, dW, db, eW, eb, mW1)


def _scale_body(ge, cell_enc, drug_enc, pgg_s, pdg_s, pcg_s, pgd_s,
                h_gg, h_dg, h_cg, h_gd):
    gene = ge[...]
    h_gg[: N_GENE, :] = gene * _norm(pgg_s[...], N_GENE)[:, None]
    h_gg[N_GENE:, :] = jnp.zeros((8, D), jnp.float32)
    h_gd[: N_GENE, :] = gene * _norm(pgd_s[...], N_GENE)[:, None]
    h_gd[N_GENE:, :] = jnp.zeros((8, D), jnp.float32)
    h_dg[: N_DRUG, :] = drug_enc[...] * _norm(pdg_s[...], N_DRUG)[:, None]
    h_dg[N_DRUG:, :] = jnp.zeros((8, D), jnp.float32)
    h_cg[: N_CELL, :] = cell_enc[...] * _norm(pcg_s[...], N_CELL)[:, None]
    h_cg[N_CELL:, :] = jnp.zeros((8, D), jnp.float32)


def _scale_h(ge, cell_enc, drug_enc, pgg_s, pdg_s, pcg_s, pgd_s):
    return pl.pallas_call(
        _scale_body,
        out_shape=[jax.ShapeDtypeStruct((HG, D), jnp.float32),
                   jax.ShapeDtypeStruct((HD, D), jnp.float32),
                   jax.ShapeDtypeStruct((HC, D), jnp.float32),
                   jax.ShapeDtypeStruct((HG, D), jnp.float32)],
    )(ge, cell_enc, drug_enc, pgg_s, pdg_s, pcg_s, pgd_s)


_ZR = 16
_SMAX = 40
_GS = 2
_KG = K // _GS


def _zero_zbuf(zbuf):
    def b(r, _):
        for j in range(8):
            zbuf[r, pl.ds(j * 16, 16)] = _Z16F()
        return ()
    lax.fori_loop(0, _ZR, b, ())


def _do_rel(c, s, src2d, dst2d, h_hbm, spmem, out_hbm,
            sstage, dstage, pays, gsems, ssems, zsem, zbuf, ep, n_row_chunks):
    zd = []
    for k in range(n_row_chunks * (K // _ZR)):
        r0 = s * n_row_chunks * K + k * _ZR
        zd.append(pltpu.async_copy(zbuf, spmem.at[pl.ds(r0, _ZR)], zsem))
    for d in zd:
        d.wait()
    plsc.subcore_barrier()

    wid = c * NS + s
    nch = ep // NW // K
    srounds = max(nch // _SMAX, 1)
    spr = nch // srounds

    def start_gather(i, b):
        return [pltpu.async_copy(
                    h_hbm.at[sstage.at[i, pl.ds(g * _KG, _KG)]],
                    pays[b].at[pl.ds(g * _KG, _KG)], gsems[b])
                for g in range(_GS)]

    for rnd in range(srounds):
        row0 = wid * nch + rnd * spr
        pltpu.sync_copy(src2d.at[pl.ds(row0, spr), :],
                        sstage.at[pl.ds(0, spr), :])
        pltpu.sync_copy(dst2d.at[pl.ds(row0, spr), :],
                        dstage.at[pl.ds(0, spr), :])
        gd = [None, None]
        sd = [None, None]
        for i in range(spr):
            b = i & 1
            if sd[b] is not None:
                sd[b].wait()
            gd[b] = start_gather(i, b)
            if i > 0:
                pb = (i - 1) & 1
                for d in gd[pb]:
                    d.wait()
                sd[pb] = pltpu.async_copy(pays[pb],
                                          spmem.at[dstage.at[i - 1]],
                                          ssems[pb], add=True)
        lb = (spr - 1) & 1
        for d in gd[lb]:
            d.wait()
        sd[lb] = pltpu.async_copy(pays[lb], spmem.at[dstage.at[spr - 1]],
                                  ssems[lb], add=True)
        for b in range(2):
            if sd[b] is not None:
                sd[b].wait()
    plsc.subcore_barrier()

    dd = []
    for k in range(n_row_chunks):
        r0 = (s * n_row_chunks + k) * K
        dd.append(pltpu.async_copy(spmem.at[pl.ds(r0, K)],
                                   out_hbm.at[c, pl.ds(r0, K)], zsem))
    for d in dd:
        d.wait()
    plsc.subcore_barrier()


def _l1_body(sgg, dgg, sdg, ddg, sgd, dgd, scg, dcg, h_gg, h_dg, h_cg, h_gd,
             p_gg, p_dg, p_cg, p_gd,
             sstage, dstage, pay0, pay1, zbuf, spm_g,
             semg0, semg1, sems0, sems1, zsem):
    c = lax.axis_index("c")
    s = lax.axis_index("s")
    pays, gsems, ssems = [pay0, pay1], [semg0, semg1], [sems0, sems1]
    _zero_zbuf(zbuf)
    _do_rel(c, s, sgd, dgd, h_gd, spm_g, p_gd, sstage, dstage, pays,
            gsems, ssems, zsem, zbuf, EP_GD, AD // (NS * K))
    _do_rel(c, s, sgg, dgg, h_gg, spm_g, p_gg, sstage, dstage, pays,
            gsems, ssems, zsem, zbuf, EP_GG, AG // (NS * K))
    _do_rel(c, s, sdg, ddg, h_dg, spm_g, p_dg, sstage, dstage, pays,
            gsems, ssems, zsem, zbuf, EP_DG, AG // (NS * K))
    _do_rel(c, s, scg, dcg, h_cg, spm_g, p_cg, sstage, dstage, pays,
            gsems, ssems, zsem, zbuf, EP_CG, AG // (NS * K))


_SC_EDGE_SCRATCH = [pltpu.VMEM((_SMAX, K), jnp.int32),
                    pltpu.VMEM((_SMAX, K), jnp.int32),
                    pltpu.VMEM((K, D), jnp.float32),
                    pltpu.VMEM((K, D), jnp.float32),
                    pltpu.VMEM((_ZR, D), jnp.float32)]

_l1_kernel = functools.partial(
    pl.kernel, _l1_body, mesh=_MESH, compiler_params=_SC_PARAMS,
    out_type=[jax.ShapeDtypeStruct((NC, AG, D), jnp.float32),
              jax.ShapeDtypeStruct((NC, AG, D), jnp.float32),
              jax.ShapeDtypeStruct((NC, AG, D), jnp.float32),
              jax.ShapeDtypeStruct((NC, AD, D), jnp.float32)],
    scratch_types=_SC_EDGE_SCRATCH
    + [pltpu.VMEM_SHARED((AG, D), jnp.float32),
       pltpu.SemaphoreType.DMA, pltpu.SemaphoreType.DMA,
       pltpu.SemaphoreType.DMA, pltpu.SemaphoreType.DMA,
       pltpu.SemaphoreType.DMA],
)()


def _l2_body(sgd, dgd, h1g, p2_gd, sstage, dstage, pay0, pay1, zbuf, spm_d,
             semg0, semg1, sems0, sems1, zsem):
    c = lax.axis_index("c")
    s = lax.axis_index("s")
    _zero_zbuf(zbuf)
    _do_rel(c, s, sgd, dgd, h1g, spm_d, p2_gd, sstage, dstage,
            [pay0, pay1], [semg0, semg1], [sems0, sems1], zsem, zbuf,
            EP_GD, AD // (NS * K))


_l2_kernel = functools.partial(
    pl.kernel, _l2_body, mesh=_MESH, compiler_params=_SC_PARAMS,
    out_type=[jax.ShapeDtypeStruct((NC, AD, D), jnp.float32)],
    scratch_types=_SC_EDGE_SCRATCH
    + [pltpu.VMEM_SHARED((AD, D), jnp.float32),
       pltpu.SemaphoreType.DMA, pltpu.SemaphoreType.DMA,
       pltpu.SemaphoreType.DMA, pltpu.SemaphoreType.DMA,
       pltpu.SemaphoreType.DMA],
)()


def _comb1_body(p_gg, p_dg, p_cg, p_gd, pgg_d, pdg_d, pcg_d, pgd_s, pgd_d,
                Wgg, bgg, Wdg, bdg, Wcg, bcg, Wgd, bgd, ge, drug_enc,
                h1g_s, h1_drug_o):
    def agg(p, n, nd_part):
        a = (p[0, :n, :] + p[1, :n, :]) * _norm(nd_part[...], n)[:, None]
        return a

    out_g = jnp.dot(agg(p_gg, N_GENE, pgg_d), Wgg[...],
                    preferred_element_type=jnp.float32) + bgg[...]
    out_g = out_g + jnp.dot(agg(p_dg, N_GENE, pdg_d), Wdg[...],
                            preferred_element_type=jnp.float32) + bdg[...]
    out_g = out_g + jnp.dot(agg(p_cg, N_GENE, pcg_d), Wcg[...],
                            preferred_element_type=jnp.float32) + bcg[...]
    h1_gene = _leaky(out_g + 0.5 * ge[...])
    h1g_s[: N_GENE, :] = h1_gene * _norm(pgd_s[...], N_GENE)[:, None]
    h1g_s[N_GENE:, :] = jnp.zeros((8, D), jnp.float32)

    out_d = jnp.dot(agg(p_gd, N_DRUG, pgd_d), Wgd[...],
                    preferred_element_type=jnp.float32) + bgd[...]
    h1_drug_o[...] = _leaky(out_d + 0.5 * drug_enc[...])


def _comb1(p_gg, p_dg, p_cg, p_gd, pgg_d, pdg_d, pcg_d, pgd_s, pgd_d,
           Wgg, bgg, Wdg, bdg, Wcg, bcg, Wgd, bgd, ge, drug_enc):
    return pl.pallas_call(
        _comb1_body,
        out_shape=[jax.ShapeDtypeStruct((HG, D), jnp.float32),
                   jax.ShapeDtypeStruct((N_DRUG, D), jnp.float32)],
    )(p_gg, p_dg, p_cg, p_gd, pgg_d, pdg_d, pcg_d, pgd_s, pgd_d,
      Wgg, bgg, Wdg, bdg, Wcg, bcg, Wgd, bgd, ge, drug_enc)


def _comb2_body(p2_gd, pgd_d, W2, b2, h1_drug, mW2, drug_mid):
    a = (p2_gd[0, :N_DRUG, :] + p2_gd[1, :N_DRUG, :]) \
        * _norm(pgd_d[...], N_DRUG)[:, None]
    h2 = _leaky(jnp.dot(a, W2[...], preferred_element_type=jnp.float32)
                + b2[...] + 0.5 * h1_drug[...])
    drug_mid[...] = jnp.dot(h2, mW2[...], preferred_element_type=jnp.float32)


def _comb2(p2_gd, pgd_d, W2, b2, h1_drug, mW2):
    return pl.pallas_call(
        _comb2_body,
        out_shape=jax.ShapeDtypeStruct((N_DRUG, MID), jnp.float32),
    )(p2_gd, pgd_d, W2, b2, h1_drug, mW2)


_HB = 64


def _head_body(cell_mid, drug_mid, cidx_hbm, didx_hbm, hv_hbm, out,
               cidx, didx, bufc, bufd, wbuf, obuf, semc, semd):
    wid = lax.axis_index("c") * NS + lax.axis_index("s")
    pltpu.sync_copy(hv_hbm, wbuf)
    for ch in range(2):
        base = wid * 128 + ch * _HB
        pltpu.sync_copy(cidx_hbm.at[pl.ds(base, _HB)], cidx)
        pltpu.sync_copy(didx_hbm.at[pl.ds(base, _HB)], didx)
        cpc = pltpu.async_copy(cell_mid.at[cidx], bufc, semc)
        cpd = pltpu.async_copy(drug_mid.at[didx], bufd, semd)
        cpc.wait()
        cpd.wait()
        ob = wbuf[pl.ds(2 * MID, 16)][0]
        lanes = lax.iota(jnp.int32, 16)

        def grp(g, _):
            def row16(k, accv):
                r = g * 16 + k
                acc = jnp.zeros((16,), jnp.float32)
                for j in range(MID // 16):
                    sl = pl.ds(j * 16, 16)
                    x = bufc[r, sl] + bufd[r, sl] + wbuf[sl]
                    x = jnp.maximum(x, 0.01 * x)
                    acc = acc + x * wbuf[pl.ds(MID + j * 16, 16)]
                return jnp.where(lanes == k, jnp.sum(acc) + ob, accv)
            vals = lax.fori_loop(0, 16, row16, jnp.zeros((16,), jnp.float32))
            obuf[pl.ds(g * 16, 16)] = vals
            return ()
        lax.fori_loop(0, _HB // 16, grp, ())
        pltpu.sync_copy(obuf, out.at[pl.ds(base, _HB)])


_head_kernel = functools.partial(
    pl.kernel, _head_body, mesh=_MESH, compiler_params=_SC_PARAMS,
    out_type=[jax.ShapeDtypeStruct((BATCH,), jnp.float32)],
    scratch_types=[pltpu.VMEM((_HB,), jnp.int32),
                   pltpu.VMEM((_HB,), jnp.int32),
                   pltpu.VMEM((_HB, MID), jnp.float32),
                   pltpu.VMEM((_HB, MID), jnp.float32),
                   pltpu.VMEM((2 * MID + 16,), jnp.float32),
                   pltpu.VMEM((_HB,), jnp.float32),
                   pltpu.SemaphoreType.DMA,
                   pltpu.SemaphoreType.DMA],
)()


def _pad_edges(ei, ns, nd, ep, spare_d):
    e = ei.shape[1]
    ar = jnp.arange(ep - e, dtype=jnp.int32)
    src = jnp.concatenate([ei[0], ns + ar % 8])
    dst = jnp.concatenate([ei[1], nd + ar % spare_d])
    return src, dst


def kernel(drug_features, cell_features, cell_index, drug_index, gene_index,
           gg_edge_index, dg_edge_index, gd_edge_index, cg_edge_index,
           gc_edge_index, params):
    p = params
    sgg, dgg = _pad_edges(gg_edge_index, N_GENE, N_GENE, EP_GG, AG - N_GENE)
    sdg, ddg = _pad_edges(dg_edge_index, N_DRUG, N_GENE, EP_DG, AG - N_GENE)
    sgd, dgd = _pad_edges(gd_edge_index, N_GENE, N_DRUG, EP_GD, AD - N_DRUG)
    scg, dcg = _pad_edges(cg_edge_index, N_CELL, N_GENE, EP_CG, AG - N_GENE)

    (pgg_s, pgg_d, pdg_s, pdg_d, pgd_s, pgd_d, pcg_s, pcg_d) = _deg_kernel(
        sgg, dgg, sdg, ddg, sgd, dgd, scg, dcg)

    cell_enc, drug_enc, cell_mid = _encoders(
        cell_features, drug_features,
        p["cell_enc_W"], p["cell_enc_b"], p["drug_enc_W"], p["drug_enc_b"],
        p["expr_enc_W"], p["expr_enc_b"], p["mid_W"][:EE])

    h_gg, h_dg, h_cg, h_gd = _scale_h(
        p["gene_emb"], cell_enc, drug_enc, pgg_s, pdg_s, pcg_s, pgd_s)

    r2 = lambda a: a.reshape(-1, K)
    p_gg, p_dg, p_cg, p_gd = _l1_kernel(
        r2(sgg), r2(dgg), r2(sdg), r2(ddg), r2(sgd), r2(dgd),
        r2(scg), r2(dcg), h_gg, h_dg, h_cg, h_gd)

    h1g_s, h1_drug = _comb1(
        p_gg, p_dg, p_cg, p_gd, pgg_d, pdg_d, pcg_d, pgd_s, pgd_d,
        p["W1_gg"], p["b1_gg"], p["W1_dg"], p["b1_dg"],
        p["W1_cg"], p["b1_cg"], p["W1_gd"], p["b1_gd"],
        p["gene_emb"], drug_enc)

    (p2_gd,) = _l2_kernel(r2(sgd), r2(dgd), h1g_s)

    drug_mid = _comb2(p2_gd, pgd_d, p["W2_gd"], p["b2_gd"], h1_drug,
                      p["mid_W"][EE:])

    hv = jnp.concatenate([p["mid_b"], p["out_W"][:, 0], p["out_b"],
                          jnp.zeros((15,), jnp.float32)])
    (out,) = _head_kernel(cell_mid, drug_mid, cell_index, drug_index, hv)
    return out.reshape(BATCH, 1)


# speedup vs baseline: 1.1376x; 1.0885x over previous
import functools

import jax
import jax.numpy as jnp
from jax import lax
from jax.experimental import pallas as pl
from jax.experimental.pallas import tpu as pltpu
from jax.experimental.pallas import tpu_sc as plsc

N_GENE, N_DRUG, N_CELL = 10000, 2000, 1000
D, EE, MID, BATCH = 128, 256, 512, 4096

NC, NS, NW = 2, 16, 32
K = 128

GP, DP, CP = 10240, 2048, 1024
AG = 10240
AD = 2048
HG, HD, HC = AG, AD, 1024

EP_GG, EP_DG, EP_GD, EP_CG = 327680, 65536, 65536, 32768

_MESH = plsc.VectorSubcoreMesh(core_axis_name="c", subcore_axis_name="s",
                               num_cores=NC, num_subcores=NS)

_Z16F = functools.partial(jnp.zeros, (16,), jnp.float32)


def _leaky(x):
    return jnp.maximum(x, 0.01 * x)


def _deg_body(sgg, dgg, sdg, ddg, sgd, dgd, scg, dcg,
              o_gg_s, o_gg_d, o_dg_s, o_dg_d, o_gd_s, o_gd_d, o_cg_s, o_cg_d,
              stage, stage2, t_gg_s, t_gg_d, t_dg_s, t_dg_d, t_gd_s, t_gd_d,
              t_cg_s, t_cg_d, ssem0, ssem1):
    wid = lax.axis_index("c") * NS + lax.axis_index("s")
    ones = jnp.full((16,), 1.0, jnp.float32)

    def zero(tbl, n):
        def b(i, _):
            tbl[pl.ds(i * 16, 16)] = _Z16F()
            return ()
        lax.fori_loop(0, n // 16, b, ())

    for tbl, n in ((t_gg_s, GP), (t_gg_d, GP), (t_dg_s, DP), (t_dg_d, GP),
                   (t_gd_s, GP), (t_gd_d, DP), (t_cg_s, CP), (t_cg_d, GP)):
        zero(tbl, n)

    work = [(sgg, t_gg_s, EP_GG), (dgg, t_gg_d, EP_GG),
            (sdg, t_dg_s, EP_DG), (ddg, t_dg_d, EP_DG),
            (sgd, t_gd_s, EP_GD), (dgd, t_gd_d, EP_GD),
            (scg, t_cg_s, EP_CG), (dcg, t_cg_d, EP_CG)]
    stages = [stage, stage2]
    sems = [ssem0, ssem1]

    def start_stage(r):
        idx_hbm, _, ep = work[r]
        epw = ep // NW
        return pltpu.async_copy(idx_hbm.at[pl.ds(wid * epw, epw)],
                                stages[r & 1].at[pl.ds(0, epw)], sems[r & 1])

    d = start_stage(0)
    for r in range(8):
        nxt = start_stage(r + 1) if r + 1 < 8 else None
        d.wait()
        _, tbl, ep = work[r]
        stg = stages[r & 1]

        def b(i, _, tbl=tbl, stg=stg):
            for u in range(4):
                v = stg[pl.ds(i * 64 + u * 16, 16)]
                plsc.addupdate_scatter(tbl, [v], ones)
            return ()
        lax.fori_loop(0, ep // NW // 64, b, ())
        d = nxt

    for tbl, out in ((t_gg_s, o_gg_s), (t_gg_d, o_gg_d), (t_dg_s, o_dg_s),
                     (t_dg_d, o_dg_d), (t_gd_s, o_gd_s), (t_gd_d, o_gd_d),
                     (t_cg_s, o_cg_s), (t_cg_d, o_cg_d)):
        pltpu.sync_copy(tbl, out.at[wid])


_SC_PARAMS = pltpu.CompilerParams(needs_layout_passes=False)

_deg_kernel = functools.partial(
    pl.kernel, _deg_body, mesh=_MESH, compiler_params=_SC_PARAMS,
    out_type=[jax.ShapeDtypeStruct((NW, GP), jnp.float32),
              jax.ShapeDtypeStruct((NW, GP), jnp.float32),
              jax.ShapeDtypeStruct((NW, DP), jnp.float32),
              jax.ShapeDtypeStruct((NW, GP), jnp.float32),
              jax.ShapeDtypeStruct((NW, GP), jnp.float32),
              jax.ShapeDtypeStruct((NW, DP), jnp.float32),
              jax.ShapeDtypeStruct((NW, CP), jnp.float32),
              jax.ShapeDtypeStruct((NW, GP), jnp.float32)],
    scratch_types=[pltpu.VMEM((EP_GG // NW,), jnp.int32),
                   pltpu.VMEM((EP_GG // NW,), jnp.int32),
                   pltpu.VMEM((GP,), jnp.float32),
                   pltpu.VMEM((GP,), jnp.float32),
                   pltpu.VMEM((DP,), jnp.float32),
                   pltpu.VMEM((GP,), jnp.float32),
                   pltpu.VMEM((GP,), jnp.float32),
                   pltpu.VMEM((DP,), jnp.float32),
                   pltpu.VMEM((CP,), jnp.float32),
                   pltpu.VMEM((GP,), jnp.float32),
                   pltpu.SemaphoreType.DMA,
                   pltpu.SemaphoreType.DMA],
)()


def _norm(partials, n):
    deg = jnp.sum(partials, axis=0)[:n]
    return lax.rsqrt(jnp.maximum(deg, 1.0))


def _enc_body(cf, df, cW, cb, dW, db, eW, eb, mW1,
              cell_enc_o, drug_enc_o, cell_mid):
    cell_enc_o[...] = _leaky(jnp.dot(cf[...], cW[...],
                                     preferred_element_type=jnp.float32)
                             + cb[...])
    drug_enc_o[...] = _leaky(jnp.dot(df[...], dW[...],
                                     preferred_element_type=jnp.float32)
                             + db[...])
    expr = _leaky(jnp.dot(cf[...], eW[...],
                          preferred_element_type=jnp.float32) + eb[...])
    cell_mid[...] = jnp.dot(expr, mW1[...], preferred_element_type=jnp.float32)


def _encoders(cf, df, cW, cb, dW, db, eW, eb, mW1):
    return pl.pallas_call(
        _enc_body,
        out_shape=[jax.ShapeDtypeStruct((N_CELL, D), jnp.float32),
                   jax.ShapeDtypeStruct((N_DRUG, D), jnp.float32),
                   jax.ShapeDtypeStruct((N_CELL, MID), jnp.float32)],
    )(cf, df, cW, cb, dW, db, eW, eb, mW1)


def _scale_body(ge, cell_enc, drug_enc, pgg_s, pdg_s, pcg_s, pgd_s,
                h_gg, h_dg, h_cg, h_gd):
    gene = ge[...]
    h_gg[: N_GENE, :] = gene * _norm(pgg_s[...], N_GENE)[:, None]
    h_gg[N_GENE:, :] = jnp.zeros((HG - N_GENE, D), jnp.float32)
    h_gd[: N_GENE, :] = gene * _norm(pgd_s[...], N_GENE)[:, None]
    h_gd[N_GENE:, :] = jnp.zeros((HG - N_GENE, D), jnp.float32)
    h_dg[: N_DRUG, :] = drug_enc[...] * _norm(pdg_s[...], N_DRUG)[:, None]
    h_dg[N_DRUG:, :] = jnp.zeros((HD - N_DRUG, D), jnp.float32)
    h_cg[: N_CELL, :] = cell_enc[...] * _norm(pcg_s[...], N_CELL)[:, None]
    h_cg[N_CELL:, :] = jnp.zeros((HC - N_CELL, D), jnp.float32)


def _scale_h(ge, cell_enc, drug_enc, pgg_s, pdg_s, pcg_s, pgd_s):
    return pl.pallas_call(
        _scale_body,
        out_shape=[jax.ShapeDtypeStruct((HG, D), jnp.float32),
                   jax.ShapeDtypeStruct((HD, D), jnp.float32),
                   jax.ShapeDtypeStruct((HC, D), jnp.float32),
                   jax.ShapeDtypeStruct((HG, D), jnp.float32)],
    )(ge, cell_enc, drug_enc, pgg_s, pdg_s, pcg_s, pgd_s)


_ZR = 16
_SMAX = 40
_GS = 2
_KG = K // _GS


def _zero_zbuf(zbuf):
    def b(r, _):
        for j in range(8):
            zbuf[r, pl.ds(j * 16, 16)] = _Z16F()
        return ()
    lax.fori_loop(0, _ZR, b, ())


def _do_rel(c, s, src2d, dst2d, h_hbm, spmem, out_hbm,
            sstage, dstage, pays, gsems, ssems, zsem, zbuf, ep, n_row_chunks):
    zd = []
    for k in range(n_row_chunks * (K // _ZR)):
        r0 = s * n_row_chunks * K + k * _ZR
        zd.append(pltpu.async_copy(zbuf, spmem.at[pl.ds(r0, _ZR)], zsem))
    for d in zd:
        d.wait()
    plsc.subcore_barrier()

    wid = c * NS + s
    nch = ep // NW // K
    srounds = max(nch // _SMAX, 1)
    spr = nch // srounds

    def start_gather(i, b):
        return [pltpu.async_copy(
                    h_hbm.at[sstage.at[i, pl.ds(g * _KG, _KG)]],
                    pays[b].at[pl.ds(g * _KG, _KG)], gsems[b])
                for g in range(_GS)]

    for rnd in range(srounds):
        row0 = wid * nch + rnd * spr
        pltpu.sync_copy(src2d.at[pl.ds(row0, spr), :],
                        sstage.at[pl.ds(0, spr), :])
        pltpu.sync_copy(dst2d.at[pl.ds(row0, spr), :],
                        dstage.at[pl.ds(0, spr), :])
        gd = [None, None]
        sd = [None, None]
        for i in range(spr):
            b = i & 1
            if sd[b] is not None:
                sd[b].wait()
            gd[b] = start_gather(i, b)
            if i > 0:
                pb = (i - 1) & 1
                for d in gd[pb]:
                    d.wait()
                sd[pb] = pltpu.async_copy(pays[pb],
                                          spmem.at[dstage.at[i - 1]],
                                          ssems[pb], add=True)
        lb = (spr - 1) & 1
        for d in gd[lb]:
            d.wait()
        sd[lb] = pltpu.async_copy(pays[lb], spmem.at[dstage.at[spr - 1]],
                                  ssems[lb], add=True)
        for b in range(2):
            if sd[b] is not None:
                sd[b].wait()
    plsc.subcore_barrier()

    dd = []
    for k in range(n_row_chunks):
        r0 = (s * n_row_chunks + k) * K
        dd.append(pltpu.async_copy(spmem.at[pl.ds(r0, K)],
                                   out_hbm.at[c, pl.ds(r0, K)], zsem))
    for d in dd:
        d.wait()
    plsc.subcore_barrier()


def _l1_body(sgg, dgg, sdg, ddg, sgd, dgd, scg, dcg, h_gg, h_dg, h_cg, h_gd,
             p_gg, p_dg, p_cg, p_gd,
             sstage, dstage, pay0, pay1, zbuf, spm_g,
             semg0, semg1, sems0, sems1, zsem):
    c = lax.axis_index("c")
    s = lax.axis_index("s")
    pays, gsems, ssems = [pay0, pay1], [semg0, semg1], [sems0, sems1]
    _zero_zbuf(zbuf)
    _do_rel(c, s, sgd, dgd, h_gd, spm_g, p_gd, sstage, dstage, pays,
            gsems, ssems, zsem, zbuf, EP_GD, AD // (NS * K))
    _do_rel(c, s, sgg, dgg, h_gg, spm_g, p_gg, sstage, dstage, pays,
            gsems, ssems, zsem, zbuf, EP_GG, AG // (NS * K))
    _do_rel(c, s, sdg, ddg, h_dg, spm_g, p_dg, sstage, dstage, pays,
            gsems, ssems, zsem, zbuf, EP_DG, AG // (NS * K))
    _do_rel(c, s, scg, dcg, h_cg, spm_g, p_cg, sstage, dstage, pays,
            gsems, ssems, zsem, zbuf, EP_CG, AG // (NS * K))


_SC_EDGE_SCRATCH = [pltpu.VMEM((_SMAX, K), jnp.int32),
                    pltpu.VMEM((_SMAX, K), jnp.int32),
                    pltpu.VMEM((K, D), jnp.float32),
                    pltpu.VMEM((K, D), jnp.float32),
                    pltpu.VMEM((_ZR, D), jnp.float32)]

_l1_kernel = functools.partial(
    pl.kernel, _l1_body, mesh=_MESH, compiler_params=_SC_PARAMS,
    out_type=[jax.ShapeDtypeStruct((NC, AG, D), jnp.float32),
              jax.ShapeDtypeStruct((NC, AG, D), jnp.float32),
              jax.ShapeDtypeStruct((NC, AG, D), jnp.float32),
              jax.ShapeDtypeStruct((NC, AD, D), jnp.float32)],
    scratch_types=_SC_EDGE_SCRATCH
    + [pltpu.VMEM_SHARED((AG, D), jnp.float32),
       pltpu.SemaphoreType.DMA, pltpu.SemaphoreType.DMA,
       pltpu.SemaphoreType.DMA, pltpu.SemaphoreType.DMA,
       pltpu.SemaphoreType.DMA],
)()


def _l2_body(sgd, dgd, h1g, p2_gd, sstage, dstage, pay0, pay1, zbuf, spm_d,
             semg0, semg1, sems0, sems1, zsem):
    c = lax.axis_index("c")
    s = lax.axis_index("s")
    _zero_zbuf(zbuf)
    _do_rel(c, s, sgd, dgd, h1g, spm_d, p2_gd, sstage, dstage,
            [pay0, pay1], [semg0, semg1], [sems0, sems1], zsem, zbuf,
            EP_GD, AD // (NS * K))


_l2_kernel = functools.partial(
    pl.kernel, _l2_body, mesh=_MESH, compiler_params=_SC_PARAMS,
    out_type=[jax.ShapeDtypeStruct((NC, AD, D), jnp.float32)],
    scratch_types=_SC_EDGE_SCRATCH
    + [pltpu.VMEM_SHARED((AD, D), jnp.float32),
       pltpu.SemaphoreType.DMA, pltpu.SemaphoreType.DMA,
       pltpu.SemaphoreType.DMA, pltpu.SemaphoreType.DMA,
       pltpu.SemaphoreType.DMA],
)()


def _comb1_body(p_gg, p_dg, p_cg, p_gd, pgg_d, pdg_d, pcg_d, pgd_s, pgd_d,
                Wgg, bgg, Wdg, bdg, Wcg, bcg, Wgd, bgd, ge, drug_enc,
                h1g_s, h1_drug_o):
    def agg(p, n, nd_part):
        a = (p[0, :n, :] + p[1, :n, :]) * _norm(nd_part[...], n)[:, None]
        return a

    out_g = jnp.dot(agg(p_gg, N_GENE, pgg_d), Wgg[...],
                    preferred_element_type=jnp.float32) + bgg[...]
    out_g = out_g + jnp.dot(agg(p_dg, N_GENE, pdg_d), Wdg[...],
                            preferred_element_type=jnp.float32) + bdg[...]
    out_g = out_g + jnp.dot(agg(p_cg, N_GENE, pcg_d), Wcg[...],
                            preferred_element_type=jnp.float32) + bcg[...]
    h1_gene = _leaky(out_g + 0.5 * ge[...])
    h1g_s[: N_GENE, :] = h1_gene * _norm(pgd_s[...], N_GENE)[:, None]
    h1g_s[N_GENE:, :] = jnp.zeros((HG - N_GENE, D), jnp.float32)

    out_d = jnp.dot(agg(p_gd, N_DRUG, pgd_d), Wgd[...],
                    preferred_element_type=jnp.float32) + bgd[...]
    h1_drug_o[...] = _leaky(out_d + 0.5 * drug_enc[...])


def _comb1(p_gg, p_dg, p_cg, p_gd, pgg_d, pdg_d, pcg_d, pgd_s, pgd_d,
           Wgg, bgg, Wdg, bdg, Wcg, bcg, Wgd, bgd, ge, drug_enc):
    return pl.pallas_call(
        _comb1_body,
        out_shape=[jax.ShapeDtypeStruct((HG, D), jnp.float32),
                   jax.ShapeDtypeStruct((N_DRUG, D), jnp.float32)],
    )(p_gg, p_dg, p_cg, p_gd, pgg_d, pdg_d, pcg_d, pgd_s, pgd_d,
      Wgg, bgg, Wdg, bdg, Wcg, bcg, Wgd, bgd, ge, drug_enc)


def _comb2_body(p2_gd, pgd_d, W2, b2, h1_drug, mW2, drug_mid):
    a = (p2_gd[0, :N_DRUG, :] + p2_gd[1, :N_DRUG, :]) \
        * _norm(pgd_d[...], N_DRUG)[:, None]
    h2 = _leaky(jnp.dot(a, W2[...], preferred_element_type=jnp.float32)
                + b2[...] + 0.5 * h1_drug[...])
    drug_mid[...] = jnp.dot(h2, mW2[...], preferred_element_type=jnp.float32)


def _comb2(p2_gd, pgd_d, W2, b2, h1_drug, mW2):
    return pl.pallas_call(
        _comb2_body,
        out_shape=jax.ShapeDtypeStruct((N_DRUG, MID), jnp.float32),
    )(p2_gd, pgd_d, W2, b2, h1_drug, mW2)


_HB = 64


def _head_body(cell_mid, drug_mid, cidx_hbm, didx_hbm, hv_hbm, out,
               cidx, didx, bufc, bufd, wbuf, obuf, semc, semd):
    wid = lax.axis_index("c") * NS + lax.axis_index("s")
    pltpu.sync_copy(hv_hbm, wbuf)
    for ch in range(2):
        base = wid * 128 + ch * _HB
        pltpu.sync_copy(cidx_hbm.at[pl.ds(base, _HB)], cidx)
        pltpu.sync_copy(didx_hbm.at[pl.ds(base, _HB)], didx)
        cpc = pltpu.async_copy(cell_mid.at[cidx], bufc, semc)
        cpd = pltpu.async_copy(drug_mid.at[didx], bufd, semd)
        cpc.wait()
        cpd.wait()
        ob = wbuf[pl.ds(2 * MID, 16)][0]
        lanes = lax.iota(jnp.int32, 16)

        def grp(g, _):
            def row16(k, accv):
                r = g * 16 + k
                acc = jnp.zeros((16,), jnp.float32)
                for j in range(MID // 16):
                    sl = pl.ds(j * 16, 16)
                    x = bufc[r, sl] + bufd[r, sl] + wbuf[sl]
                    x = jnp.maximum(x, 0.01 * x)
                    acc = acc + x * wbuf[pl.ds(MID + j * 16, 16)]
                return jnp.where(lanes == k, jnp.sum(acc) + ob, accv)
            vals = lax.fori_loop(0, 16, row16, jnp.zeros((16,), jnp.float32))
            obuf[pl.ds(g * 16, 16)] = vals
            return ()
        lax.fori_loop(0, _HB // 16, grp, ())
        pltpu.sync_copy(obuf, out.at[pl.ds(base, _HB)])


_head_kernel = functools.partial(
    pl.kernel, _head_body, mesh=_MESH, compiler_params=_SC_PARAMS,
    out_type=[jax.ShapeDtypeStruct((BATCH,), jnp.float32)],
    scratch_types=[pltpu.VMEM((_HB,), jnp.int32),
                   pltpu.VMEM((_HB,), jnp.int32),
                   pltpu.VMEM((_HB, MID), jnp.float32),
                   pltpu.VMEM((_HB, MID), jnp.float32),
                   pltpu.VMEM((2 * MID + 16,), jnp.float32),
                   pltpu.VMEM((_HB,), jnp.float32),
                   pltpu.SemaphoreType.DMA,
                   pltpu.SemaphoreType.DMA],
)()


def _pad_edges(ei, ns, nd, ep, spare_s, spare_d):
    e = ei.shape[1]
    ar = jnp.arange(ep - e, dtype=jnp.int32)
    src = jnp.concatenate([ei[0], ns + ar % spare_s])
    dst = jnp.concatenate([ei[1], nd + ar % spare_d])
    return src, dst


def kernel(drug_features, cell_features, cell_index, drug_index, gene_index,
           gg_edge_index, dg_edge_index, gd_edge_index, cg_edge_index,
           gc_edge_index, params):
    p = params
    sgg, dgg = _pad_edges(gg_edge_index, N_GENE, N_GENE, EP_GG,
                          HG - N_GENE, AG - N_GENE)
    sdg, ddg = _pad_edges(dg_edge_index, N_DRUG, N_GENE, EP_DG,
                          HD - N_DRUG, AG - N_GENE)
    sgd, dgd = _pad_edges(gd_edge_index, N_GENE, N_DRUG, EP_GD,
                          HG - N_GENE, AD - N_DRUG)
    scg, dcg = _pad_edges(cg_edge_index, N_CELL, N_GENE, EP_CG,
                          HC - N_CELL, AG - N_GENE)

    (pgg_s, pgg_d, pdg_s, pdg_d, pgd_s, pgd_d, pcg_s, pcg_d) = _deg_kernel(
        sgg, dgg, sdg, ddg, sgd, dgd, scg, dcg)

    cell_enc, drug_enc, cell_mid = _encoders(
        cell_features, drug_features,
        p["cell_enc_W"], p["cell_enc_b"], p["drug_enc_W"], p["drug_enc_b"],
        p["expr_enc_W"], p["expr_enc_b"], p["mid_W"][:EE])

    h_gg, h_dg, h_cg, h_gd = _scale_h(
        p["gene_emb"], cell_enc, drug_enc, pgg_s, pdg_s, pcg_s, pgd_s)

    r2 = lambda a: a.reshape(-1, K)
    p_gg, p_dg, p_cg, p_gd = _l1_kernel(
        r2(sgg), r2(dgg), r2(sdg), r2(ddg), r2(sgd), r2(dgd),
        r2(scg), r2(dcg), h_gg, h_dg, h_cg, h_gd)

    h1g_s, h1_drug = _comb1(
        p_gg, p_dg, p_cg, p_gd, pgg_d, pdg_d, pcg_d, pgd_s, pgd_d,
        p["W1_gg"], p["b1_gg"], p["W1_dg"], p["b1_dg"],
        p["W1_cg"], p["b1_cg"], p["W1_gd"], p["b1_gd"],
        p["gene_emb"], drug_enc)

    (p2_gd,) = _l2_kernel(r2(sgd), r2(dgd), h1g_s)

    drug_mid = _comb2(p2_gd, pgd_d, p["W2_gd"], p["b2_gd"], h1_drug,
                      p["mid_W"][EE:])

    hv = jnp.concatenate([p["mid_b"], p["out_W"][:, 0], p["out_b"],
                          jnp.zeros((15,), jnp.float32)])
    (out,) = _head_kernel(cell_mid, drug_mid, cell_index, drug_index, hv)
    return out.reshape(BATCH, 1)


# speedup vs baseline: 1.1487x; 1.0098x over previous
import functools

import jax
import jax.numpy as jnp
from jax import lax
from jax.experimental import pallas as pl
from jax.experimental.pallas import tpu as pltpu
from jax.experimental.pallas import tpu_sc as plsc

N_GENE, N_DRUG, N_CELL = 10000, 2000, 1000
D, EE, MID, BATCH = 128, 256, 512, 4096

NC, NS, NW = 2, 16, 32
K = 128

GP, DP, CP = 10240, 2048, 1024
AG = 10240
AD = 2048
HG, HD, HC = AG, AD, 1024

EP_GG, EP_DG, EP_GD, EP_CG = 327680, 65536, 65536, 32768

_MESH = plsc.VectorSubcoreMesh(core_axis_name="c", subcore_axis_name="s",
                               num_cores=NC, num_subcores=NS)

_Z16F = functools.partial(jnp.zeros, (16,), jnp.float32)


def _leaky(x):
    return jnp.maximum(x, 0.01 * x)


def _deg_body(sgg, dgg, sdg, ddg, sgd, dgd, scg, dcg,
              o_gg_s, o_gg_d, o_dg_s, o_dg_d, o_gd_s, o_gd_d, o_cg_s, o_cg_d,
              stage, stage2, t_gg_s, t_gg_d, t_dg_s, t_dg_d, t_gd_s, t_gd_d,
              t_cg_s, t_cg_d, ssem0, ssem1):
    wid = lax.axis_index("c") * NS + lax.axis_index("s")
    ones = jnp.full((16,), 1.0, jnp.float32)

    def zero(tbl, n):
        def b(i, _):
            tbl[pl.ds(i * 16, 16)] = _Z16F()
            return ()
        lax.fori_loop(0, n // 16, b, ())

    for tbl, n in ((t_gg_s, GP), (t_gg_d, GP), (t_dg_s, DP), (t_dg_d, GP),
                   (t_gd_s, GP), (t_gd_d, DP), (t_cg_s, CP), (t_cg_d, GP)):
        zero(tbl, n)

    work = [(sgg, t_gg_s, EP_GG), (dgg, t_gg_d, EP_GG),
            (sdg, t_dg_s, EP_DG), (ddg, t_dg_d, EP_DG),
            (sgd, t_gd_s, EP_GD), (dgd, t_gd_d, EP_GD),
            (scg, t_cg_s, EP_CG), (dcg, t_cg_d, EP_CG)]
    stages = [stage, stage2]
    sems = [ssem0, ssem1]

    def start_stage(r):
        idx_hbm, _, ep = work[r]
        epw = ep // NW
        return pltpu.async_copy(idx_hbm.at[pl.ds(wid * epw, epw)],
                                stages[r & 1].at[pl.ds(0, epw)], sems[r & 1])

    d = start_stage(0)
    for r in range(8):
        nxt = start_stage(r + 1) if r + 1 < 8 else None
        d.wait()
        _, tbl, ep = work[r]
        stg = stages[r & 1]

        def b(i, _, tbl=tbl, stg=stg):
            for u in range(4):
                v = stg[pl.ds(i * 64 + u * 16, 16)]
                plsc.addupdate_scatter(tbl, [v], ones)
            return ()
        lax.fori_loop(0, ep // NW // 64, b, ())
        d = nxt

    for tbl, out in ((t_gg_s, o_gg_s), (t_gg_d, o_gg_d), (t_dg_s, o_dg_s),
                     (t_dg_d, o_dg_d), (t_gd_s, o_gd_s), (t_gd_d, o_gd_d),
                     (t_cg_s, o_cg_s), (t_cg_d, o_cg_d)):
        pltpu.sync_copy(tbl, out.at[wid])


_SC_PARAMS = pltpu.CompilerParams(needs_layout_passes=False)

_deg_kernel = functools.partial(
    pl.kernel, _deg_body, mesh=_MESH, compiler_params=_SC_PARAMS,
    out_type=[jax.ShapeDtypeStruct((NW, GP), jnp.float32),
              jax.ShapeDtypeStruct((NW, GP), jnp.float32),
              jax.ShapeDtypeStruct((NW, DP), jnp.float32),
              jax.ShapeDtypeStruct((NW, GP), jnp.float32),
              jax.ShapeDtypeStruct((NW, GP), jnp.float32),
              jax.ShapeDtypeStruct((NW, DP), jnp.float32),
              jax.ShapeDtypeStruct((NW, CP), jnp.float32),
              jax.ShapeDtypeStruct((NW, GP), jnp.float32)],
    scratch_types=[pltpu.VMEM((EP_GG // NW,), jnp.int32),
                   pltpu.VMEM((EP_GG // NW,), jnp.int32),
                   pltpu.VMEM((GP,), jnp.float32),
                   pltpu.VMEM((GP,), jnp.float32),
                   pltpu.VMEM((DP,), jnp.float32),
                   pltpu.VMEM((GP,), jnp.float32),
                   pltpu.VMEM((GP,), jnp.float32),
                   pltpu.VMEM((DP,), jnp.float32),
                   pltpu.VMEM((CP,), jnp.float32),
                   pltpu.VMEM((GP,), jnp.float32),
                   pltpu.SemaphoreType.DMA,
                   pltpu.SemaphoreType.DMA],
)()


def _norm(partials, n):
    deg = jnp.sum(partials, axis=0)[:n]
    return lax.rsqrt(jnp.maximum(deg, 1.0))


def _enc_body(cf, df, cW, cb, dW, db, eW, eb, mW1,
              cell_enc_o, drug_enc_o, cell_mid):
    cell_enc_o[...] = _leaky(jnp.dot(cf[...], cW[...],
                                     preferred_element_type=jnp.float32)
                             + cb[...])
    drug_enc_o[...] = _leaky(jnp.dot(df[...], dW[...],
                                     preferred_element_type=jnp.float32)
                             + db[...])
    expr = _leaky(jnp.dot(cf[...], eW[...],
                          preferred_element_type=jnp.float32) + eb[...])
    cell_mid[...] = jnp.dot(expr, mW1[...], preferred_element_type=jnp.float32)


def _encoders(cf, df, cW, cb, dW, db, eW, eb, mW1):
    return pl.pallas_call(
        _enc_body,
        out_shape=[jax.ShapeDtypeStruct((N_CELL, D), jnp.float32),
                   jax.ShapeDtypeStruct((N_DRUG, D), jnp.float32),
                   jax.ShapeDtypeStruct((N_CELL, MID), jnp.float32)],
    )(cf, df, cW, cb, dW, db, eW, eb, mW1)


def _scale_body(ge, cell_enc, drug_enc, pgg_s, pdg_s, pcg_s, pgd_s,
                h_gg, h_dg, h_cg, h_gd):
    gene = ge[...]
    h_gg[: N_GENE, :] = gene * _norm(pgg_s[...], N_GENE)[:, None]
    h_gg[N_GENE:, :] = jnp.zeros((HG - N_GENE, D), jnp.float32)
    h_gd[: N_GENE, :] = gene * _norm(pgd_s[...], N_GENE)[:, None]
    h_gd[N_GENE:, :] = jnp.zeros((HG - N_GENE, D), jnp.float32)
    h_dg[: N_DRUG, :] = drug_enc[...] * _norm(pdg_s[...], N_DRUG)[:, None]
    h_dg[N_DRUG:, :] = jnp.zeros((HD - N_DRUG, D), jnp.float32)
    h_cg[: N_CELL, :] = cell_enc[...] * _norm(pcg_s[...], N_CELL)[:, None]
    h_cg[N_CELL:, :] = jnp.zeros((HC - N_CELL, D), jnp.float32)


def _scale_h(ge, cell_enc, drug_enc, pgg_s, pdg_s, pcg_s, pgd_s):
    return pl.pallas_call(
        _scale_body,
        out_shape=[jax.ShapeDtypeStruct((HG, D), jnp.float32),
                   jax.ShapeDtypeStruct((HD, D), jnp.float32),
                   jax.ShapeDtypeStruct((HC, D), jnp.float32),
                   jax.ShapeDtypeStruct((HG, D), jnp.float32)],
    )(ge, cell_enc, drug_enc, pgg_s, pdg_s, pcg_s, pgd_s)


_ZR = 16
_SMAX = 40
_GS = 2
_KG = K // _GS


def _zero_zbuf(zbuf):
    def b(r, _):
        for j in range(8):
            zbuf[r, pl.ds(j * 16, 16)] = _Z16F()
        return ()
    lax.fori_loop(0, _ZR, b, ())


def _do_rel(c, s, src2d, dst2d, h_hbm, spmem, out_hbm,
            sstage, dstage, pays, gsems, ssems, zsem, zbuf, ep, n_row_chunks):
    zd = []
    for k in range(n_row_chunks * (K // _ZR)):
        r0 = s * n_row_chunks * K + k * _ZR
        zd.append(pltpu.async_copy(zbuf, spmem.at[pl.ds(r0, _ZR)], zsem))
    for d in zd:
        d.wait()
    plsc.subcore_barrier()

    wid = c * NS + s
    nch = ep // NW // K
    srounds = max(nch // _SMAX, 1)
    spr = nch // srounds

    def start_gather(i, b):
        return [pltpu.async_copy(
                    h_hbm.at[sstage.at[i, pl.ds(g * _KG, _KG)]],
                    pays[b].at[pl.ds(g * _KG, _KG)], gsems[b])
                for g in range(_GS)]

    for rnd in range(srounds):
        row0 = wid * nch + rnd * spr
        pltpu.sync_copy(src2d.at[pl.ds(row0, spr), :],
                        sstage.at[pl.ds(0, spr), :])
        pltpu.sync_copy(dst2d.at[pl.ds(row0, spr), :],
                        dstage.at[pl.ds(0, spr), :])
        gd = [None, None]
        sd = [None, None]
        for i in range(spr):
            b = i & 1
            if sd[b] is not None:
                sd[b].wait()
            gd[b] = start_gather(i, b)
            if i > 0:
                pb = (i - 1) & 1
                for d in gd[pb]:
                    d.wait()
                sd[pb] = pltpu.async_copy(pays[pb],
                                          spmem.at[dstage.at[i - 1]],
                                          ssems[pb], add=True)
        lb = (spr - 1) & 1
        for d in gd[lb]:
            d.wait()
        sd[lb] = pltpu.async_copy(pays[lb], spmem.at[dstage.at[spr - 1]],
                                  ssems[lb], add=True)
        for b in range(2):
            if sd[b] is not None:
                sd[b].wait()
    plsc.subcore_barrier()

    dd = []
    for k in range(n_row_chunks):
        r0 = (s * n_row_chunks + k) * K
        dd.append(pltpu.async_copy(spmem.at[pl.ds(r0, K)],
                                   out_hbm.at[c, pl.ds(r0, K)], zsem))
    for d in dd:
        d.wait()
    plsc.subcore_barrier()


def _l1_body(sgg, dgg, sdg, ddg, sgd, dgd, scg, dcg, h_gg, h_dg, h_cg, h_gd,
             p_gg, p_dg, p_cg, p_gd,
             sstage, dstage, pay0, pay1, zbuf, spm_g,
             semg0, semg1, sems0, sems1, zsem):
    c = lax.axis_index("c")
    s = lax.axis_index("s")
    pays, gsems, ssems = [pay0, pay1], [semg0, semg1], [sems0, sems1]
    _zero_zbuf(zbuf)
    _do_rel(c, s, sgd, dgd, h_gd, spm_g, p_gd, sstage, dstage, pays,
            gsems, ssems, zsem, zbuf, EP_GD, AD // (NS * K))
    _do_rel(c, s, sgg, dgg, h_gg, spm_g, p_gg, sstage, dstage, pays,
            gsems, ssems, zsem, zbuf, EP_GG, AG // (NS * K))
    _do_rel(c, s, sdg, ddg, h_dg, spm_g, p_dg, sstage, dstage, pays,
            gsems, ssems, zsem, zbuf, EP_DG, AG // (NS * K))
    _do_rel(c, s, scg, dcg, h_cg, spm_g, p_cg, sstage, dstage, pays,
            gsems, ssems, zsem, zbuf, EP_CG, AG // (NS * K))


_SC_EDGE_SCRATCH = [pltpu.VMEM((_SMAX, K), jnp.int32),
                    pltpu.VMEM((_SMAX, K), jnp.int32),
                    pltpu.VMEM((K, D), jnp.float32),
                    pltpu.VMEM((K, D), jnp.float32),
                    pltpu.VMEM((_ZR, D), jnp.float32)]

_l1_kernel = functools.partial(
    pl.kernel, _l1_body, mesh=_MESH, compiler_params=_SC_PARAMS,
    out_type=[jax.ShapeDtypeStruct((NC, AG, D), jnp.float32),
              jax.ShapeDtypeStruct((NC, AG, D), jnp.float32),
              jax.ShapeDtypeStruct((NC, AG, D), jnp.float32),
              jax.ShapeDtypeStruct((NC, AD, D), jnp.float32)],
    scratch_types=_SC_EDGE_SCRATCH
    + [pltpu.VMEM_SHARED((AG, D), jnp.float32),
       pltpu.SemaphoreType.DMA, pltpu.SemaphoreType.DMA,
       pltpu.SemaphoreType.DMA, pltpu.SemaphoreType.DMA,
       pltpu.SemaphoreType.DMA],
)()


def _l2_body(sgd, dgd, h1g, p2_gd, sstage, dstage, pay0, pay1, zbuf, spm_d,
             semg0, semg1, sems0, sems1, zsem):
    c = lax.axis_index("c")
    s = lax.axis_index("s")
    _zero_zbuf(zbuf)
    _do_rel(c, s, sgd, dgd, h1g, spm_d, p2_gd, sstage, dstage,
            [pay0, pay1], [semg0, semg1], [sems0, sems1], zsem, zbuf,
            EP_GD, AD // (NS * K))


_l2_kernel = functools.partial(
    pl.kernel, _l2_body, mesh=_MESH, compiler_params=_SC_PARAMS,
    out_type=[jax.ShapeDtypeStruct((NC, AD, D), jnp.float32)],
    scratch_types=_SC_EDGE_SCRATCH
    + [pltpu.VMEM_SHARED((AD, D), jnp.float32),
       pltpu.SemaphoreType.DMA, pltpu.SemaphoreType.DMA,
       pltpu.SemaphoreType.DMA, pltpu.SemaphoreType.DMA,
       pltpu.SemaphoreType.DMA],
)()


_RB = 1280


def _aggn(p, nd_part):
    deg = jnp.sum(nd_part[...], axis=0)
    nd = lax.rsqrt(jnp.maximum(deg, 1.0))
    return (p[0] + p[1]) * nd[:, None]


def _comb1g_body(p_gg, p_dg, p_cg, pgg_d, pdg_d, pcg_d, pgd_s, ge,
                 Wgg, bgg, Wdg, bdg, Wcg, bcg, h1g_s):
    out_g = jnp.dot(_aggn(p_gg, pgg_d), Wgg[...],
                    preferred_element_type=jnp.float32) + bgg[...]
    out_g = out_g + jnp.dot(_aggn(p_dg, pdg_d), Wdg[...],
                            preferred_element_type=jnp.float32) + bdg[...]
    out_g = out_g + jnp.dot(_aggn(p_cg, pcg_d), Wcg[...],
                            preferred_element_type=jnp.float32) + bcg[...]
    h1_gene = _leaky(out_g + 0.5 * ge[...])
    deg = jnp.sum(pgd_s[...], axis=0)
    h1g_s[...] = h1_gene * lax.rsqrt(jnp.maximum(deg, 1.0))[:, None]


def _comb1g(p_gg, p_dg, p_cg, pgg_d, pdg_d, pcg_d, pgd_s, ge_pad,
            Wgg, bgg, Wdg, bdg, Wcg, bcg):
    pblk = pl.BlockSpec((NC, _RB, D), lambda i: (0, i, 0))
    dblk = pl.BlockSpec((NW, _RB), lambda i: (0, i))
    wblk = pl.BlockSpec((D, D), lambda i: (0, 0))
    bblk = pl.BlockSpec((D,), lambda i: (0,))
    return pl.pallas_call(
        _comb1g_body,
        grid=(HG // _RB,),
        in_specs=[pblk, pblk, pblk, dblk, dblk, dblk, dblk,
                  pl.BlockSpec((_RB, D), lambda i: (i, 0)),
                  wblk, bblk, wblk, bblk, wblk, bblk],
        out_specs=pl.BlockSpec((_RB, D), lambda i: (i, 0)),
        out_shape=jax.ShapeDtypeStruct((HG, D), jnp.float32),
    )(p_gg, p_dg, p_cg, pgg_d, pdg_d, pcg_d, pgd_s, ge_pad,
      Wgg, bgg, Wdg, bdg, Wcg, bcg)


def _comb1d_body(p_gd, pgd_d, Wgd, bgd, drug_enc, h1_drug_o):
    a = (p_gd[0, :N_DRUG, :] + p_gd[1, :N_DRUG, :]) \
        * _norm(pgd_d[...], N_DRUG)[:, None]
    out_d = jnp.dot(a, Wgd[...], preferred_element_type=jnp.float32) + bgd[...]
    h1_drug_o[...] = _leaky(out_d + 0.5 * drug_enc[...])


def _comb1d(p_gd, pgd_d, Wgd, bgd, drug_enc):
    return pl.pallas_call(
        _comb1d_body,
        out_shape=jax.ShapeDtypeStruct((N_DRUG, D), jnp.float32),
    )(p_gd, pgd_d, Wgd, bgd, drug_enc)


def _comb2_body(p2_gd, pgd_d, W2, b2, h1_drug, mW2, drug_mid):
    a = (p2_gd[0, :N_DRUG, :] + p2_gd[1, :N_DRUG, :]) \
        * _norm(pgd_d[...], N_DRUG)[:, None]
    h2 = _leaky(jnp.dot(a, W2[...], preferred_element_type=jnp.float32)
                + b2[...] + 0.5 * h1_drug[...])
    drug_mid[...] = jnp.dot(h2, mW2[...], preferred_element_type=jnp.float32)


def _comb2(p2_gd, pgd_d, W2, b2, h1_drug, mW2):
    return pl.pallas_call(
        _comb2_body,
        out_shape=jax.ShapeDtypeStruct((N_DRUG, MID), jnp.float32),
    )(p2_gd, pgd_d, W2, b2, h1_drug, mW2)


_HB = 64


def _head_body(cell_mid, drug_mid, cidx_hbm, didx_hbm, hv_hbm, out,
               cidx, didx, bufc, bufd, wbuf, obuf, semc, semd):
    wid = lax.axis_index("c") * NS + lax.axis_index("s")
    pltpu.sync_copy(hv_hbm, wbuf)
    for ch in range(2):
        base = wid * 128 + ch * _HB
        pltpu.sync_copy(cidx_hbm.at[pl.ds(base, _HB)], cidx)
        pltpu.sync_copy(didx_hbm.at[pl.ds(base, _HB)], didx)
        cpc = pltpu.async_copy(cell_mid.at[cidx], bufc, semc)
        cpd = pltpu.async_copy(drug_mid.at[didx], bufd, semd)
        cpc.wait()
        cpd.wait()
        ob = wbuf[pl.ds(2 * MID, 16)][0]
        lanes = lax.iota(jnp.int32, 16)

        def grp(g, _):
            def row16(k, accv):
                r = g * 16 + k
                acc = jnp.zeros((16,), jnp.float32)
                for j in range(MID // 16):
                    sl = pl.ds(j * 16, 16)
                    x = bufc[r, sl] + bufd[r, sl] + wbuf[sl]
                    x = jnp.maximum(x, 0.01 * x)
                    acc = acc + x * wbuf[pl.ds(MID + j * 16, 16)]
                return jnp.where(lanes == k, jnp.sum(acc) + ob, accv)
            vals = lax.fori_loop(0, 16, row16, jnp.zeros((16,), jnp.float32))
            obuf[pl.ds(g * 16, 16)] = vals
            return ()
        lax.fori_loop(0, _HB // 16, grp, ())
        pltpu.sync_copy(obuf, out.at[pl.ds(base, _HB)])


_head_kernel = functools.partial(
    pl.kernel, _head_body, mesh=_MESH, compiler_params=_SC_PARAMS,
    out_type=[jax.ShapeDtypeStruct((BATCH,), jnp.float32)],
    scratch_types=[pltpu.VMEM((_HB,), jnp.int32),
                   pltpu.VMEM((_HB,), jnp.int32),
                   pltpu.VMEM((_HB, MID), jnp.float32),
                   pltpu.VMEM((_HB, MID), jnp.float32),
                   pltpu.VMEM((2 * MID + 16,), jnp.float32),
                   pltpu.VMEM((_HB,), jnp.float32),
                   pltpu.SemaphoreType.DMA,
                   pltpu.SemaphoreType.DMA],
)()


def _pad_edges(ei, ns, nd, ep, spare_s, spare_d):
    e = ei.shape[1]
    ar = jnp.arange(ep - e, dtype=jnp.int32)
    src = jnp.concatenate([ei[0], ns + ar % spare_s])
    dst = jnp.concatenate([ei[1], nd + ar % spare_d])
    return src, dst


def kernel(drug_features, cell_features, cell_index, drug_index, gene_index,
           gg_edge_index, dg_edge_index, gd_edge_index, cg_edge_index,
           gc_edge_index, params):
    p = params
    sgg, dgg = _pad_edges(gg_edge_index, N_GENE, N_GENE, EP_GG,
                          HG - N_GENE, AG - N_GENE)
    sdg, ddg = _pad_edges(dg_edge_index, N_DRUG, N_GENE, EP_DG,
                          HD - N_DRUG, AG - N_GENE)
    sgd, dgd = _pad_edges(gd_edge_index, N_GENE, N_DRUG, EP_GD,
                          HG - N_GENE, AD - N_DRUG)
    scg, dcg = _pad_edges(cg_edge_index, N_CELL, N_GENE, EP_CG,
                          HC - N_CELL, AG - N_GENE)

    (pgg_s, pgg_d, pdg_s, pdg_d, pgd_s, pgd_d, pcg_s, pcg_d) = _deg_kernel(
        sgg, dgg, sdg, ddg, sgd, dgd, scg, dcg)

    cell_enc, drug_enc, cell_mid = _encoders(
        cell_features, drug_features,
        p["cell_enc_W"], p["cell_enc_b"], p["drug_enc_W"], p["drug_enc_b"],
        p["expr_enc_W"], p["expr_enc_b"], p["mid_W"][:EE])

    h_gg, h_dg, h_cg, h_gd = _scale_h(
        p["gene_emb"], cell_enc, drug_enc, pgg_s, pdg_s, pcg_s, pgd_s)

    r2 = lambda a: a.reshape(-1, K)
    p_gg, p_dg, p_cg, p_gd = _l1_kernel(
        r2(sgg), r2(dgg), r2(sdg), r2(ddg), r2(sgd), r2(dgd),
        r2(scg), r2(dcg), h_gg, h_dg, h_cg, h_gd)

    ge_pad = jnp.concatenate([p["gene_emb"],
                              jnp.zeros((HG - N_GENE, D), jnp.float32)])
    h1g_s = _comb1g(p_gg, p_dg, p_cg, pgg_d, pdg_d, pcg_d, pgd_s, ge_pad,
                    p["W1_gg"], p["b1_gg"], p["W1_dg"], p["b1_dg"],
                    p["W1_cg"], p["b1_cg"])
    h1_drug = _comb1d(p_gd, pgd_d, p["W1_gd"], p["b1_gd"], drug_enc)

    (p2_gd,) = _l2_kernel(r2(sgd), r2(dgd), h1g_s)

    drug_mid = _comb2(p2_gd, pgd_d, p["W2_gd"], p["b2_gd"], h1_drug,
                      p["mid_W"][EE:])

    hv = jnp.concatenate([p["mid_b"], p["out_W"][:, 0], p["out_b"],
                          jnp.zeros((15,), jnp.float32)])
    (out,) = _head_kernel(cell_mid, drug_mid, cell_index, drug_index, hv)
    return out.reshape(BATCH, 1)
